# baseline (device time: 25631 ns/iter reference)
import jax
import jax.numpy as jnp
from jax import lax
from jax.experimental import pallas as pl
from jax.experimental.pallas import tpu as pltpu

N_DEV = 4
SUB = 2
_GELU_C = 0.7978845608028654


def _gelu(y):
    return 0.5 * y * (1.0 + jnp.tanh(_GELU_C * (y + 0.044715 * y * y * y)))


def kernel(x, w_mat):
    m_per, k = x.shape
    _, n_per = w_mat.shape
    k2 = k // 2
    m_sub = m_per // SUB

    def body(x_ref, w_ref, out_ref, nbr_l, nbr_r, diag_a, diag_b,
             send_sems, recv_sems):
        my = lax.axis_index("i")
        left = lax.rem(my + N_DEV - 1, N_DEV)
        right = lax.rem(my + 1, N_DEV)

        barrier_sem = pltpu.get_barrier_semaphore()
        for nbr in (left, right):
            pl.semaphore_signal(
                barrier_sem, inc=1,
                device_id=(nbr,), device_id_type=pl.DeviceIdType.MESH,
            )
        pl.semaphore_wait(barrier_sem, 2)

        def rowsub(origin, s):
            return pl.ds(
                lax.rem(origin + N_DEV, N_DEV) * m_per + s * m_sub, m_sub)

        def dot(a, b):
            return jnp.dot(a, b, preferred_element_type=jnp.float32)

        def mk(t, s):
            rows = pl.ds(s * m_sub, m_sub)
            src, dst, dev = [
                (x_ref.at[rows, :], nbr_l.at[rows, :], right),
                (x_ref.at[rows, :], nbr_r.at[rows, :], left),
                (nbr_l.at[rows, pl.ds(0, k2)], diag_a.at[rows, :], right),
                (nbr_r.at[rows, pl.ds(k2, k2)], diag_b.at[rows, :], left),
            ][t]
            return pltpu.make_async_remote_copy(
                src_ref=src, dst_ref=dst,
                send_sem=send_sems.at[t, s],
                recv_sem=recv_sems.at[t, s],
                device_id=(dev,), device_id_type=pl.DeviceIdType.MESH,
            )

        descs = {}
        for s in range(SUB):
            for t in (0, 1):
                r = mk(t, s)
                r.start()
                descs[(t, s)] = r

        out_ref[pl.ds(my * m_per, m_per), :] = _gelu(
            dot(x_ref[...], w_ref[...]))

        for s in range(SUB):
            descs[(0, s)].wait_recv()
            r = mk(2, s)
            r.start()
            descs[(2, s)] = r
            descs[(1, s)].wait_recv()
            r = mk(3, s)
            r.start()
            descs[(3, s)] = r
            rows = pl.ds(s * m_sub, m_sub)
            out_ref[rowsub(my - 1, s), :] = _gelu(
                dot(nbr_l[rows, :], w_ref[...]))
            out_ref[rowsub(my + 1, s), :] = _gelu(
                dot(nbr_r[rows, :], w_ref[...]))

        w_top = w_ref[:k2, :]
        w_bot = w_ref[k2:, :]
        for s in range(SUB):
            descs[(2, s)].wait_recv()
            descs[(3, s)].wait_recv()
            rows = pl.ds(s * m_sub, m_sub)
            out_ref[rowsub(my + 2, s), :] = _gelu(
                dot(diag_a[rows, :], w_top) + dot(diag_b[rows, :], w_bot))

        for r in descs.values():
            r.wait_send()

    return pl.pallas_call(
        body,
        out_shape=jax.ShapeDtypeStruct((N_DEV * m_per, n_per), jnp.float32),
        in_specs=[
            pl.BlockSpec(memory_space=pltpu.VMEM),
            pl.BlockSpec(memory_space=pltpu.VMEM),
        ],
        out_specs=pl.BlockSpec(memory_space=pltpu.VMEM),
        scratch_shapes=[
            pltpu.VMEM((m_per, k), jnp.float32),
            pltpu.VMEM((m_per, k), jnp.float32),
            pltpu.VMEM((m_per, k2), jnp.float32),
            pltpu.VMEM((m_per, k2), jnp.float32),
            pltpu.SemaphoreType.DMA((4, SUB)),
            pltpu.SemaphoreType.DMA((4, SUB)),
        ],
        compiler_params=pltpu.CompilerParams(collective_id=0),
    )(x, w_mat)
